# baseline (speedup 1.0000x reference)
import jax
import jax.numpy as jnp
from jax.experimental import pallas as pl

_N = 10000
_BLK = 1000


def _mlp_block(x_ref, w1_ref, b1_ref, w2_ref, b2_ref, w3_ref, b3_ref, out_ref):
    h = jnp.dot(x_ref[...], w1_ref[...], preferred_element_type=jnp.float32)
    h = jnp.maximum(h + b1_ref[...], 0.0)
    h = jnp.dot(h, w2_ref[...], preferred_element_type=jnp.float32)
    h = jnp.maximum(h + b2_ref[...], 0.0)
    logits = jnp.dot(h, w3_ref[...], preferred_element_type=jnp.float32)
    logits = logits + b3_ref[...]
    m = jnp.max(logits, axis=1, keepdims=True)
    e = jnp.exp(logits - m)
    out_ref[...] = e / jnp.sum(e, axis=1, keepdims=True)


def kernel(x, edge_index, edge_weight, W1, b1, W2, b2, W3, b3):
    del edge_index, edge_weight
    f_in = x.shape[1]
    c = W2.shape[0]
    n_cls = W3.shape[1]

    grid = (_N // _BLK,)
    fixed = lambda i: (0, 0)
    fixed1 = lambda i: (0,)
    out = pl.pallas_call(
        _mlp_block,
        grid=grid,
        in_specs=[
            pl.BlockSpec((_BLK, f_in), lambda i: (i, 0)),
            pl.BlockSpec((f_in, c), fixed),
            pl.BlockSpec((c,), fixed1),
            pl.BlockSpec((c, c), fixed),
            pl.BlockSpec((c,), fixed1),
            pl.BlockSpec((c, n_cls), fixed),
            pl.BlockSpec((n_cls,), fixed1),
        ],
        out_specs=pl.BlockSpec((_BLK, n_cls), lambda i: (i, 0)),
        out_shape=jax.ShapeDtypeStruct((_N, n_cls), jnp.float32),
    )(x, W1, b1, W2, b2, W3, b3)
    return out


# speedup vs baseline: 1.2992x; 1.2992x over previous
import jax
import jax.numpy as jnp
from jax.experimental import pallas as pl

_N = 10000
_BLK = 5000


def _mlp_block(x_ref, w1_ref, b1_ref, w2_ref, b2_ref, w3_ref, b3_ref, out_ref):
    h = jnp.dot(x_ref[...], w1_ref[...], preferred_element_type=jnp.float32)
    h = jnp.maximum(h + b1_ref[...], 0.0)
    h = jnp.dot(h, w2_ref[...], preferred_element_type=jnp.float32)
    h = jnp.maximum(h + b2_ref[...], 0.0)
    logits = jnp.dot(h, w3_ref[...], preferred_element_type=jnp.float32)
    logits = logits + b3_ref[...]
    m = jnp.max(logits, axis=1, keepdims=True)
    e = jnp.exp(logits - m)
    out_ref[...] = e / jnp.sum(e, axis=1, keepdims=True)


def kernel(x, edge_index, edge_weight, W1, b1, W2, b2, W3, b3):
    del edge_index, edge_weight
    f_in = x.shape[1]
    c = W2.shape[0]
    n_cls = W3.shape[1]

    grid = (_N // _BLK,)
    fixed = lambda i: (0, 0)
    fixed1 = lambda i: (0,)
    out = pl.pallas_call(
        _mlp_block,
        grid=grid,
        in_specs=[
            pl.BlockSpec((_BLK, f_in), lambda i: (i, 0)),
            pl.BlockSpec((f_in, c), fixed),
            pl.BlockSpec((c,), fixed1),
            pl.BlockSpec((c, c), fixed),
            pl.BlockSpec((c,), fixed1),
            pl.BlockSpec((c, n_cls), fixed),
            pl.BlockSpec((n_cls,), fixed1),
        ],
        out_specs=pl.BlockSpec((_BLK, n_cls), lambda i: (i, 0)),
        out_shape=jax.ShapeDtypeStruct((_N, n_cls), jnp.float32),
    )(x, W1, b1, W2, b2, W3, b3)
    return out
